# Initial kernel scaffold; baseline (speedup 1.0000x reference)
import jax
import jax.numpy as jnp
from jax.experimental import pallas as pl


def kernel(input_ids, token_type_ids, word_emb, pos_emb, type_emb, ln_weight, ln_bias):
    raise NotImplementedError("write your pallas kernel here")


# speedup vs baseline: 1.3408x; 1.3408x over previous
import functools

import jax
import jax.numpy as jnp
from jax import lax
from jax.experimental import pallas as pl
from jax.experimental.pallas import tpu as pltpu
from jax.experimental.pallas import tpu_sc as plsc

NC = 2
NS = 16
NW = NC * NS
L = 16
D = 128
DV = D // L
T = 128
EPS = 1e-12


def _build(N, S, V):
    ntok = N // NW
    nchunks = ntok // T
    mesh = plsc.VectorSubcoreMesh(core_axis_name="c", subcore_axis_name="s")

    @functools.partial(
        pl.kernel,
        out_type=jax.ShapeDtypeStruct((N, D), jnp.float32),
        mesh=mesh,
        compiler_params=pltpu.CompilerParams(needs_layout_passes=False),
        scratch_types=[
            pltpu.VMEM((T,), jnp.int32),
            pltpu.VMEM((T,), jnp.int32),
            pltpu.VMEM((T, D), jnp.float32),
            pltpu.VMEM((2 * S, D), jnp.float32),
            pltpu.VMEM((2, D), jnp.float32),
            pltpu.VMEM((D,), jnp.float32),
            pltpu.VMEM((D,), jnp.float32),
            pltpu.VMEM((T,), jnp.float32),
            pltpu.VMEM((T,), jnp.float32),
            pltpu.SemaphoreType.DMA,
        ],
    )
    def k(ids_hbm, tts_hbm, wemb_hbm, pemb_hbm, temb_hbm, lnw_hbm, lnb_hbm,
          out_hbm, idx_v, tt_v, wrows, ptblk, tvb, lnw_v, lnb_v, ub, rb, sem):
        wid = lax.axis_index("s") * NC + lax.axis_index("c")
        base_w = wid * ntok
        iota16 = lax.broadcasted_iota(jnp.int32, (L,), 0)

        pltpu.sync_copy(pemb_hbm.at[pl.ds(0, S)], ptblk.at[pl.ds(0, S)])
        pltpu.sync_copy(pemb_hbm.at[pl.ds(0, S)], ptblk.at[pl.ds(S, S)])
        pltpu.sync_copy(temb_hbm, tvb)
        pltpu.sync_copy(lnw_hbm, lnw_v)
        pltpu.sync_copy(lnb_hbm, lnb_v)

        t0 = [tvb[0, pl.ds(j * L, L)] for j in range(DV)]
        t1 = [tvb[1, pl.ds(j * L, L)] for j in range(DV)]

        def add_type(p, carry):
            for j in range(DV):
                sl = pl.ds(j * L, L)
                ptblk[p, sl] = ptblk[p, sl] + t0[j]
                ptblk[S + p, sl] = ptblk[S + p, sl] + t1[j]
            return carry

        lax.fori_loop(0, S, add_type, 0)

        lnw_r = [lnw_v[pl.ds(j * L, L)] for j in range(DV)]
        lnb_r = [lnb_v[pl.ds(j * L, L)] for j in range(DV)]
        zero = jnp.zeros((L,), jnp.float32)

        def chunk_body(c, carry):
            base = base_w + c * T
            pltpu.sync_copy(ids_hbm.at[pl.ds(base, T)], idx_v)
            pltpu.async_copy(wemb_hbm.at[idx_v], wrows, sem).wait()
            pltpu.sync_copy(tts_hbm.at[pl.ds(base, T)], tt_v)

            for g in range(T // L):
                tt = tt_v[pl.ds(g * L, L)]
                pos = lax.rem(c * T + g * L + iota16, S)
                ptrow = tt * S + pos
                tokidx = iota16 + g * L

                def d_body(d, sq):
                    s, q = sq
                    dd = jnp.full((L,), d, jnp.int32)
                    w = plsc.load_gather(wrows, [tokidx, dd])
                    p = plsc.load_gather(ptblk, [ptrow, dd])
                    a = w + p
                    plsc.store_scatter(wrows, [tokidx, dd], a)
                    return (s + a, q + a * a)

                s, q = lax.fori_loop(0, D, d_body, (zero, zero))
                u = s * (1.0 / D)
                var = jnp.maximum(q * (1.0 / D) - u * u, 0.0) + EPS
                vi = lax.bitcast_convert_type(var, jnp.int32)
                yi = jnp.int32(0x5F3759DF) - lax.shift_right_logical(
                    vi, jnp.int32(1))
                y = lax.bitcast_convert_type(yi, jnp.float32)
                for _ in range(3):
                    y = y * (1.5 - 0.5 * var * y * y)
                ub[pl.ds(g * L, L)] = u
                rb[pl.ds(g * L, L)] = y
            def t_body(t, carry2):
                tvec = jnp.full((L,), t, jnp.int32)
                uu = plsc.load_gather(ub, [tvec])
                rr = plsc.load_gather(rb, [tvec])
                for j in range(DV):
                    sl = pl.ds(j * L, L)
                    a = wrows[t, sl]
                    wrows[t, sl] = (a - uu) * rr * lnw_r[j] + lnb_r[j]
                return carry2

            lax.fori_loop(0, T, t_body, 0)
            pltpu.sync_copy(wrows, out_hbm.at[pl.ds(base, T)])
            return carry

        lax.fori_loop(0, nchunks, chunk_body, 0)

    return k


def kernel(input_ids, token_type_ids, word_emb, pos_emb, type_emb,
           ln_weight, ln_bias):
    B, S = input_ids.shape
    V, d = word_emb.shape
    N = B * S
    ids = input_ids.reshape(N).astype(jnp.int32)
    tts = token_type_ids.reshape(N).astype(jnp.int32)
    k = _build(N, S, V)
    out = k(ids, tts, word_emb, pos_emb, type_emb,
            ln_weight.astype(jnp.float32), ln_bias.astype(jnp.float32))
    return out.reshape(B, S, d)


# speedup vs baseline: 4.9418x; 3.6856x over previous
import functools

import jax
import jax.numpy as jnp
from jax import lax
from jax.experimental import pallas as pl
from jax.experimental.pallas import tpu as pltpu
from jax.experimental.pallas import tpu_sc as plsc

NC = 2
NS = 16
NW = NC * NS
L = 16
D = 128
DV = D // L
T = 128
EPS = 1e-12


def _build(N, S, V):
    ntok = N // NW
    nchunks = ntok // T
    mesh = plsc.VectorSubcoreMesh(core_axis_name="c", subcore_axis_name="s")

    @functools.partial(
        pl.kernel,
        out_type=jax.ShapeDtypeStruct((N, D), jnp.float32),
        mesh=mesh,
        compiler_params=pltpu.CompilerParams(needs_layout_passes=False),
        scratch_types=[
            pltpu.VMEM((T,), jnp.int32),
            pltpu.VMEM((T,), jnp.int32),
            pltpu.VMEM((T,), jnp.int32),
            pltpu.VMEM((T,), jnp.int32),
            pltpu.VMEM((T, D), jnp.float32),
            pltpu.VMEM((T, D), jnp.float32),
            pltpu.VMEM((S, D), jnp.float32),
            pltpu.VMEM((2, D), jnp.float32),
            pltpu.VMEM((D,), jnp.float32),
            pltpu.VMEM((D,), jnp.float32),
            pltpu.SemaphoreType.DMA,
            pltpu.SemaphoreType.DMA,
            pltpu.SemaphoreType.DMA,
            pltpu.SemaphoreType.DMA,
        ],
    )
    def k(ids_hbm, tts_hbm, wemb_hbm, pemb_hbm, temb_hbm, lnw_hbm, lnb_hbm,
          out_hbm, idx0, idx1, tt0, tt1, wr0, wr1, posb, tvb, lnw_v, lnb_v,
          gs0, gs1, os0, os1):
        idxv = (idx0, idx1)
        ttv = (tt0, tt1)
        wr = (wr0, wr1)
        gsem = (gs0, gs1)
        osem = (os0, os1)

        wid = lax.axis_index("s") * NC + lax.axis_index("c")
        base_w = wid * ntok

        pltpu.sync_copy(pemb_hbm.at[pl.ds(0, S)], posb)
        pltpu.sync_copy(temb_hbm, tvb)
        pltpu.sync_copy(lnw_hbm, lnw_v)
        pltpu.sync_copy(lnb_hbm, lnb_v)

        t0r = [tvb[0, pl.ds(j * L, L)] for j in range(DV)]
        dtr = [tvb[1, pl.ds(j * L, L)] - t0r[j] for j in range(DV)]
        lnw_r = [lnw_v[pl.ds(j * L, L)] for j in range(DV)]
        lnb_r = [lnb_v[pl.ds(j * L, L)] for j in range(DV)]

        pltpu.sync_copy(ids_hbm.at[pl.ds(base_w, T)], idxv[0])
        pltpu.sync_copy(tts_hbm.at[pl.ds(base_w, T)], ttv[0])
        pltpu.async_copy(wemb_hbm.at[idxv[0]], wr[0], gsem[0])

        def compute_chunk(i, p):
            buf = wr[p]
            ttb = ttv[p]

            def t_body(t, carry):
                pos = lax.rem(i * T + t, S)
                ttf = plsc.load_gather(
                    ttb, [jnp.full((L,), t, jnp.int32)]).astype(jnp.float32)
                a = []
                s_acc = None
                q_acc = None
                for j in range(DV):
                    sl = pl.ds(j * L, L)
                    x = buf[t, sl] + posb[pos, sl] + t0r[j] + ttf * dtr[j]
                    a.append(x)
                    s_acc = x if s_acc is None else s_acc + x
                    q_acc = x * x if q_acc is None else q_acc + x * x
                s = jnp.sum(s_acc)
                q = jnp.sum(q_acc)
                uu = jnp.full((L,), s, jnp.float32) * (1.0 / D)
                qq = jnp.full((L,), q, jnp.float32) * (1.0 / D)
                var = jnp.maximum(qq - uu * uu, 0.0) + EPS
                vi = lax.bitcast_convert_type(var, jnp.int32)
                yi = jnp.int32(0x5F3759DF) - lax.shift_right_logical(
                    vi, jnp.int32(1))
                y = lax.bitcast_convert_type(yi, jnp.float32)
                for _ in range(3):
                    y = y * (1.5 - 0.5 * var * y * y)
                for j in range(DV):
                    sl = pl.ds(j * L, L)
                    buf[t, sl] = (a[j] - uu) * y * lnw_r[j] + lnb_r[j]
                return carry

            lax.fori_loop(0, T, t_body, 0, unroll=2)

        def step(i, p, q):
            base = base_w + i * T
            pltpu.make_async_copy(wemb_hbm.at[idxv[p]], wr[p], gsem[p]).wait()

            @pl.when(i + 1 < nchunks)
            def _():
                @pl.when(i >= 1)
                def _():
                    pltpu.make_async_copy(
                        wr[q], out_hbm.at[pl.ds(base_w, T)], osem[q]).wait()

                nbase = base + T
                pltpu.sync_copy(ids_hbm.at[pl.ds(nbase, T)], idxv[q])
                pltpu.sync_copy(tts_hbm.at[pl.ds(nbase, T)], ttv[q])
                pltpu.async_copy(wemb_hbm.at[idxv[q]], wr[q], gsem[q])

            compute_chunk(i, p)
            pltpu.async_copy(wr[p], out_hbm.at[pl.ds(base, T)], osem[p])

        def pair_body(h, carry):
            step(2 * h, 0, 1)
            step(2 * h + 1, 1, 0)
            return carry

        lax.fori_loop(0, nchunks // 2, pair_body, 0)
        pltpu.make_async_copy(wr[0], out_hbm.at[pl.ds(base_w, T)], osem[0]).wait()
        pltpu.make_async_copy(wr[1], out_hbm.at[pl.ds(base_w, T)], osem[1]).wait()

    return k


def kernel(input_ids, token_type_ids, word_emb, pos_emb, type_emb,
           ln_weight, ln_bias):
    B, S = input_ids.shape
    V, d = word_emb.shape
    N = B * S
    ids = input_ids.reshape(N).astype(jnp.int32)
    tts = token_type_ids.reshape(N).astype(jnp.int32)
    k = _build(N, S, V)
    out = k(ids, tts, word_emb, pos_emb, type_emb,
            ln_weight.astype(jnp.float32), ln_bias.astype(jnp.float32))
    return out.reshape(B, S, d)


# speedup vs baseline: 14.0535x; 2.8438x over previous
import functools

import jax
import jax.numpy as jnp
from jax import lax
from jax.experimental import pallas as pl
from jax.experimental.pallas import tpu as pltpu
from jax.experimental.pallas import tpu_sc as plsc

NC = 2
NS = 16
NW = NC * NS
L = 16
D = 128
DV = D // L
T = 128
EPS = 1e-12


def _build(N, S, V):
    ntok = N // NW
    nchunks = ntok // T
    mesh = plsc.VectorSubcoreMesh(core_axis_name="c", subcore_axis_name="s")

    @functools.partial(
        pl.kernel,
        out_type=jax.ShapeDtypeStruct((N, D), jnp.float32),
        mesh=mesh,
        compiler_params=pltpu.CompilerParams(needs_layout_passes=False),
        scratch_types=[
            pltpu.VMEM((T,), jnp.int32),
            pltpu.VMEM((T,), jnp.int32),
            pltpu.VMEM((T,), jnp.int32),
            pltpu.VMEM((T,), jnp.int32),
            pltpu.VMEM((T, D), jnp.float32),
            pltpu.VMEM((T, D), jnp.float32),
            pltpu.VMEM((T, D), jnp.float32),
            pltpu.VMEM((T, D), jnp.float32),
            pltpu.VMEM((S, D), jnp.float32),
            pltpu.VMEM((2, D), jnp.float32),
            pltpu.VMEM((D,), jnp.float32),
            pltpu.VMEM((D,), jnp.float32),
            pltpu.SemaphoreType.DMA,
            pltpu.SemaphoreType.DMA,
            pltpu.SemaphoreType.DMA,
            pltpu.SemaphoreType.DMA,
        ],
    )
    def k(ids_hbm, tts_hbm, wemb_hbm, pemb_hbm, temb_hbm, lnw_hbm, lnb_hbm,
          out_hbm, idx0, idx1, tt0, tt1, wr0, wr1, ob0, ob1, posb, tvb, lnw_v, lnb_v,
          gs0, gs1, os0, os1):
        idxv = (idx0, idx1)
        ttv = (tt0, tt1)
        wr = (wr0, wr1)
        obuf = (ob0, ob1)
        gsem = (gs0, gs1)
        osem = (os0, os1)

        wid = lax.axis_index("s") * NC + lax.axis_index("c")
        base_w = wid * ntok

        pltpu.sync_copy(pemb_hbm.at[pl.ds(0, S)], posb)
        pltpu.sync_copy(temb_hbm, tvb)
        pltpu.sync_copy(lnw_hbm, lnw_v)
        pltpu.sync_copy(lnb_hbm, lnb_v)

        t0r = [tvb[0, pl.ds(j * L, L)] for j in range(DV)]
        dtr = [tvb[1, pl.ds(j * L, L)] - t0r[j] for j in range(DV)]
        lnw_r = [lnw_v[pl.ds(j * L, L)] for j in range(DV)]
        lnb_r = [lnb_v[pl.ds(j * L, L)] for j in range(DV)]

        pltpu.sync_copy(ids_hbm.at[pl.ds(base_w, T)], idxv[0])
        pltpu.sync_copy(tts_hbm.at[pl.ds(base_w, T)], ttv[0])
        pltpu.async_copy(wemb_hbm.at[idxv[0]], wr[0], gsem[0])

        def compute_chunk(i, p):
            buf = wr[p]
            ob = obuf[p]
            ttb = ttv[p]

            @functools.partial(plsc.parallel_loop, 0, T, unroll=4)
            def t_body(t):
                pos = lax.rem(i * T + t, S)
                ttf = plsc.load_gather(
                    ttb, [jnp.full((L,), t, jnp.int32)]).astype(jnp.float32)
                a = []
                s_acc = None
                q_acc = None
                for j in range(DV):
                    sl = pl.ds(j * L, L)
                    x = buf[t, sl] + posb[pos, sl] + t0r[j] + ttf * dtr[j]
                    a.append(x)
                    s_acc = x if s_acc is None else s_acc + x
                    q_acc = x * x if q_acc is None else q_acc + x * x
                s = jnp.sum(s_acc)
                q = jnp.sum(q_acc)
                uu = jnp.full((L,), s, jnp.float32) * (1.0 / D)
                qq = jnp.full((L,), q, jnp.float32) * (1.0 / D)
                var = jnp.maximum(qq - uu * uu, 0.0) + EPS
                vi = lax.bitcast_convert_type(var, jnp.int32)
                yi = jnp.int32(0x5F3759DF) - lax.shift_right_logical(
                    vi, jnp.int32(1))
                y = lax.bitcast_convert_type(yi, jnp.float32)
                for _ in range(3):
                    y = y * (1.5 - 0.5 * var * y * y)
                for j in range(DV):
                    sl = pl.ds(j * L, L)
                    ob[t, sl] = (a[j] - uu) * y * lnw_r[j] + lnb_r[j]

        def step(i, p, q):
            base = base_w + i * T
            pltpu.make_async_copy(wemb_hbm.at[idxv[p]], wr[p], gsem[p]).wait()

            @pl.when(i + 1 < nchunks)
            def _():
                nbase = base + T
                pltpu.sync_copy(ids_hbm.at[pl.ds(nbase, T)], idxv[q])
                pltpu.sync_copy(tts_hbm.at[pl.ds(nbase, T)], ttv[q])
                pltpu.async_copy(wemb_hbm.at[idxv[q]], wr[q], gsem[q])

            @pl.when(i >= 2)
            def _():
                pltpu.make_async_copy(
                    obuf[p], out_hbm.at[pl.ds(base_w, T)], osem[p]).wait()

            compute_chunk(i, p)
            pltpu.async_copy(obuf[p], out_hbm.at[pl.ds(base, T)], osem[p])

        def pair_body(h, carry):
            step(2 * h, 0, 1)
            step(2 * h + 1, 1, 0)
            return carry

        lax.fori_loop(0, nchunks // 2, pair_body, 0)
        pltpu.make_async_copy(obuf[0], out_hbm.at[pl.ds(base_w, T)], osem[0]).wait()
        pltpu.make_async_copy(obuf[1], out_hbm.at[pl.ds(base_w, T)], osem[1]).wait()

    return k


def kernel(input_ids, token_type_ids, word_emb, pos_emb, type_emb,
           ln_weight, ln_bias):
    B, S = input_ids.shape
    V, d = word_emb.shape
    N = B * S
    ids = input_ids.reshape(N).astype(jnp.int32)
    tts = token_type_ids.reshape(N).astype(jnp.int32)
    k = _build(N, S, V)
    out = k(ids, tts, word_emb, pos_emb, type_emb,
            ln_weight.astype(jnp.float32), ln_bias.astype(jnp.float32))
    return out.reshape(B, S, d)
